# Initial kernel scaffold; baseline (speedup 1.0000x reference)
import jax
import jax.numpy as jnp
from jax.experimental import pallas as pl


def kernel(constraint_features, edge_index, edge_attr, variable_features, Wc, bc, Wv, bv, We, be, Wmc, Wmv, ln1_g, ln1_b, ln2_g, ln2_b, ln3_g, ln3_b, fc1_w, fc1_b, fc2_w, fc2_b, fc3_w, fc3_b, fc4_w, fc4_b, fc5_w, fc5_b):
    raise NotImplementedError("write your pallas kernel here")


# speedup vs baseline: 3.0549x; 3.0549x over previous
import functools

import jax
import jax.numpy as jnp
from jax import lax
from jax.experimental import pallas as pl
from jax.experimental.pallas import tpu as pltpu
from jax.experimental.pallas import tpu_sc as plsc

N_CONS = 10000
N_VARS = 10000
N_EDGES = 320000
D_FEAT = 128
D_EDGE = 16
H = 64

NC = 2
NS = 16
NW = NC * NS
EB = 128
N_ROWS = 2528
ROWS_PER_W = N_ROWS // NW
E_PAD = N_ROWS * EB
N_SEG_PAD = 10240
ROWS_PER_TILE = N_SEG_PAD // NS


def _node_embed_kernel(x_ref, w_ref, b_ref, o_ref):
    o_ref[...] = jnp.maximum(
        jnp.dot(x_ref[...], w_ref[...], preferred_element_type=jnp.float32)
        + b_ref[...], 0.0)


def _node_embed(x, w, b):
    n = x.shape[0]
    blk = n // 5
    return pl.pallas_call(
        _node_embed_kernel,
        grid=(5,),
        in_specs=[
            pl.BlockSpec((blk, D_FEAT), lambda i: (i, 0)),
            pl.BlockSpec((D_FEAT, H), lambda i: (0, 0)),
            pl.BlockSpec((1, H), lambda i: (0, 0)),
        ],
        out_specs=pl.BlockSpec((blk, H), lambda i: (i, 0)),
        out_shape=jax.ShapeDtypeStruct((n, H), jnp.float32),
    )(x, w, b.reshape(1, H))


_EBLK = E_PAD // 16


def _edge_embed_kernel(x_ref, w_ref, b_ref, o_ref):
    y = jnp.maximum(
        jnp.dot(x_ref[...], w_ref[...], preferred_element_type=jnp.float32)
        + b_ref[...], 0.0)
    row = pl.program_id(0) * _EBLK + lax.broadcasted_iota(
        jnp.int32, (_EBLK, 1), 0)
    o_ref[...] = jnp.where(row < N_EDGES, y, 0.0)


def _edge_embed(ea_pad, we, be):
    return pl.pallas_call(
        _edge_embed_kernel,
        grid=(16,),
        in_specs=[
            pl.BlockSpec((_EBLK, D_EDGE), lambda i: (i, 0)),
            pl.BlockSpec((D_EDGE, H), lambda i: (0, 0)),
            pl.BlockSpec((1, H), lambda i: (0, 0)),
        ],
        out_specs=pl.BlockSpec((_EBLK, H), lambda i: (i, 0)),
        out_shape=jax.ShapeDtypeStruct((E_PAD, H), jnp.float32),
    )(ea_pad, we, be.reshape(1, H))


def _update_kernel(x_ref, w_ref, b_ref, parts_ref, wm_ref, o_ref):
    m = parts_ref[0] + parts_ref[1]
    h0 = jnp.dot(x_ref[...], w_ref[...], preferred_element_type=jnp.float32)
    h0 = jnp.maximum(h0 + b_ref[...], 0.0)
    upd = jnp.dot(m, wm_ref[...], preferred_element_type=jnp.float32)
    o_ref[...] = jnp.maximum(h0 + upd, 0.0)


def _update(x, w, b, parts, wm):
    n = x.shape[0]
    blk = n // 5
    return pl.pallas_call(
        _update_kernel,
        grid=(5,),
        in_specs=[
            pl.BlockSpec((blk, D_FEAT), lambda i: (i, 0)),
            pl.BlockSpec((D_FEAT, H), lambda i: (0, 0)),
            pl.BlockSpec((1, H), lambda i: (0, 0)),
            pl.BlockSpec((2, blk, H), lambda i: (0, i, 0)),
            pl.BlockSpec((H, H), lambda i: (0, 0)),
        ],
        out_specs=pl.BlockSpec((blk, H), lambda i: (i, 0)),
        out_shape=jax.ShapeDtypeStruct((n, H), jnp.float32),
    )(x, w, b.reshape(1, H), parts, wm)


def _ln(x, g, b, eps=1e-5):
    mu = jnp.mean(x, axis=-1, keepdims=True)
    d = x - mu
    var = jnp.mean(d * d, axis=-1, keepdims=True)
    return d * lax.rsqrt(var + eps) * g + b


def _final_kernel(hv0_ref, parts_ref, wmv_ref,
                  ln1g, ln1b, ln2g, ln2b, ln3g, ln3b,
                  f1w, f1b, f2w, f2b, f3w, f3b, f4w, f4b, f5w, f5b,
                  o_ref):
    m = parts_ref[0, :N_VARS] + parts_ref[1, :N_VARS]
    hv = jnp.maximum(
        hv0_ref[...]
        + jnp.dot(m, wmv_ref[...], preferred_element_type=jnp.float32), 0.0)
    x = jnp.mean(hv, axis=0, keepdims=True)
    x = _ln(x, ln1g[...], ln1b[...])
    x = jnp.maximum(jnp.dot(x, f1w[...], preferred_element_type=jnp.float32)
                    + f1b[...], 0.0)
    x = _ln(x, ln2g[...], ln2b[...])
    x = jnp.maximum(jnp.dot(x, f2w[...], preferred_element_type=jnp.float32)
                    + f2b[...], 0.0)
    x = jnp.dot(x, f3w[...], preferred_element_type=jnp.float32) + f3b[...]
    x = jnp.maximum(_ln(x, ln3g[...], ln3b[...]), 0.0)
    x = jnp.maximum(jnp.dot(x, f4w[...], preferred_element_type=jnp.float32)
                    + f4b[...], 0.0)
    o_ref[...] = jnp.dot(x, f5w[...], preferred_element_type=jnp.float32) \
        + f5b[...]


def _final(hv0, parts, wmv, head):
    args = [hv0, parts, wmv] + head
    return pl.pallas_call(
        _final_kernel,
        out_shape=jax.ShapeDtypeStruct((1, 1), jnp.float32),
    )(*args)


def _sc_pass(table, gidx, sidx, e3d, zeros_tbl):
    mesh = plsc.VectorSubcoreMesh(core_axis_name="c", subcore_axis_name="s")
    rpt = ROWS_PER_TILE

    @functools.partial(
        pl.kernel,
        out_type=jax.ShapeDtypeStruct((NC, N_SEG_PAD, H), jnp.float32),
        mesh=mesh,
        scratch_types=[
            pltpu.VMEM((EB,), jnp.int32),
            pltpu.VMEM((EB,), jnp.int32),
            pltpu.VMEM((EB, H), jnp.float32),
            pltpu.VMEM((EB, H), jnp.float32),
            pltpu.VMEM_SHARED((N_SEG_PAD, H), jnp.float32),
            pltpu.SemaphoreType.DMA,
        ],
        compiler_params=pltpu.CompilerParams(use_tc_tiling_on_sc=False),
    )
    def k(table_hbm, gidx_hbm, sidx_hbm, e_hbm, zeros_hbm, out_hbm,
          gi_v, si_v, rows_v, e_v, acc, sem):
        cid = lax.axis_index("c")
        sid = lax.axis_index("s")
        wid = cid * NS + sid
        pltpu.sync_copy(zeros_hbm.at[pl.ds(sid * rpt, rpt)],
                        acc.at[pl.ds(sid * rpt, rpt)])
        plsc.subcore_barrier()

        def body(j, carry):
            r = wid * ROWS_PER_W + j
            pltpu.sync_copy(gidx_hbm.at[pl.ds(r * EB, EB)], gi_v)
            pltpu.sync_copy(sidx_hbm.at[pl.ds(r * EB, EB)], si_v)
            pltpu.async_copy(table_hbm.at[gi_v], rows_v, sem).wait()
            pltpu.sync_copy(e_hbm.at[r], e_v)

            def mul_row(q, c2):
                for c in range(H // 16):
                    s = pl.ds(c * 16, 16)
                    rows_v[q, s] = rows_v[q, s] * e_v[q, s]
                return c2

            lax.fori_loop(0, EB, mul_row, 0)
            pltpu.sync_copy(rows_v, acc.at[si_v], add=True)
            return carry

        lax.fori_loop(0, ROWS_PER_W, body, 0)
        plsc.subcore_barrier()
        pltpu.sync_copy(acc.at[pl.ds(sid * rpt, rpt)],
                        out_hbm.at[cid, pl.ds(sid * rpt, rpt)])

    return k(table, gidx, sidx, e3d, zeros_tbl)


def _pad_rows(x, n_rows):
    return jnp.concatenate(
        [x, jnp.zeros((n_rows - x.shape[0],) + x.shape[1:], x.dtype)])


def kernel(constraint_features, edge_index, edge_attr, variable_features,
           Wc, bc, Wv, bv, We, be, Wmc, Wmv,
           ln1_g, ln1_b, ln2_g, ln2_b, ln3_g, ln3_b,
           fc1_w, fc1_b, fc2_w, fc2_b, fc3_w, fc3_b,
           fc4_w, fc4_b, fc5_w, fc5_b):
    src_p = _pad_rows(edge_index[0].astype(jnp.int32), E_PAD)
    dst_p = _pad_rows(edge_index[1].astype(jnp.int32), E_PAD)
    ea_pad = _pad_rows(edge_attr, E_PAD)
    zeros_tbl = jnp.zeros((N_SEG_PAD, H), jnp.float32)

    h_v0 = _node_embed(variable_features, Wv, bv)
    e = _edge_embed(ea_pad, We, be).reshape(N_ROWS, EB, H)

    msgc_parts = _sc_pass(h_v0, dst_p, src_p, e, zeros_tbl)
    h_c = _update(constraint_features, Wc, bc, msgc_parts, Wmc)

    msgv_parts = _sc_pass(h_c, src_p, dst_p, e, zeros_tbl)

    head = [ln1_g.reshape(1, H), ln1_b.reshape(1, H),
            ln2_g.reshape(1, 128), ln2_b.reshape(1, 128),
            ln3_g.reshape(1, 256), ln3_b.reshape(1, 256),
            fc1_w, fc1_b.reshape(1, 128), fc2_w, fc2_b.reshape(1, 128),
            fc3_w, fc3_b.reshape(1, 256), fc4_w, fc4_b.reshape(1, 128),
            fc5_w, fc5_b.reshape(1, 1)]
    out = _final(h_v0, msgv_parts, Wmv, head)
    return out.reshape(1)


# speedup vs baseline: 5.9232x; 1.9389x over previous
import functools

import jax
import jax.numpy as jnp
from jax import lax
from jax.experimental import pallas as pl
from jax.experimental.pallas import tpu as pltpu
from jax.experimental.pallas import tpu_sc as plsc

N_CONS = 10000
N_VARS = 10000
N_EDGES = 320000
D_FEAT = 128
D_EDGE = 16
H = 64

NC = 2
NS = 16
NW = NC * NS
EB = 128
N_BLK = N_EDGES // EB
N_ROWS = 2528
ROWS_PER_W = N_ROWS // NW
E_PAD = N_ROWS * EB
N_SEG_PAD = 10240
ROWS_PER_TILE = N_SEG_PAD // NS


def _node_embed_kernel(x_ref, w_ref, b_ref, o_ref):
    o_ref[...] = jnp.maximum(
        jnp.dot(x_ref[...], w_ref[...], preferred_element_type=jnp.float32)
        + b_ref[...], 0.0)


def _node_embed(x, w, b):
    n = x.shape[0]
    blk = n // 5
    return pl.pallas_call(
        _node_embed_kernel,
        grid=(5,),
        in_specs=[
            pl.BlockSpec((blk, D_FEAT), lambda i: (i, 0)),
            pl.BlockSpec((D_FEAT, H), lambda i: (0, 0)),
            pl.BlockSpec((1, H), lambda i: (0, 0)),
        ],
        out_specs=pl.BlockSpec((blk, H), lambda i: (i, 0)),
        out_shape=jax.ShapeDtypeStruct((n, H), jnp.float32),
    )(x, w, b.reshape(1, H))


_EBLK = N_EDGES // 16


def _edge_embed_kernel(x_ref, w_ref, b_ref, o_ref):
    o_ref[...] = jnp.maximum(
        jnp.dot(x_ref[...], w_ref[...], preferred_element_type=jnp.float32)
        + b_ref[...], 0.0)


def _edge_embed(ea, we, be):
    return pl.pallas_call(
        _edge_embed_kernel,
        grid=(16,),
        in_specs=[
            pl.BlockSpec((_EBLK, D_EDGE), lambda i: (i, 0)),
            pl.BlockSpec((D_EDGE, H), lambda i: (0, 0)),
            pl.BlockSpec((1, H), lambda i: (0, 0)),
        ],
        out_specs=pl.BlockSpec((_EBLK, H), lambda i: (i, 0)),
        out_shape=jax.ShapeDtypeStruct((N_EDGES, H), jnp.float32),
    )(ea, we, be.reshape(1, H))


def _update_kernel(x_ref, w_ref, b_ref, parts_ref, wm_ref, o_ref):
    m = parts_ref[0] + parts_ref[1]
    h0 = jnp.dot(x_ref[...], w_ref[...], preferred_element_type=jnp.float32)
    h0 = jnp.maximum(h0 + b_ref[...], 0.0)
    upd = jnp.dot(m, wm_ref[...], preferred_element_type=jnp.float32)
    o_ref[...] = jnp.maximum(h0 + upd, 0.0)


def _update(x, w, b, parts, wm):
    n = x.shape[0]
    blk = n // 5
    return pl.pallas_call(
        _update_kernel,
        grid=(5,),
        in_specs=[
            pl.BlockSpec((blk, D_FEAT), lambda i: (i, 0)),
            pl.BlockSpec((D_FEAT, H), lambda i: (0, 0)),
            pl.BlockSpec((1, H), lambda i: (0, 0)),
            pl.BlockSpec((2, blk, H), lambda i: (0, i, 0)),
            pl.BlockSpec((H, H), lambda i: (0, 0)),
        ],
        out_specs=pl.BlockSpec((blk, H), lambda i: (i, 0)),
        out_shape=jax.ShapeDtypeStruct((n, H), jnp.float32),
    )(x, w, b.reshape(1, H), parts, wm)


def _ln(x, g, b, eps=1e-5):
    mu = jnp.mean(x, axis=-1, keepdims=True)
    d = x - mu
    var = jnp.mean(d * d, axis=-1, keepdims=True)
    return d * lax.rsqrt(var + eps) * g + b


def _final_kernel(hv0_ref, parts_ref, wmv_ref,
                  ln1g, ln1b, ln2g, ln2b, ln3g, ln3b,
                  f1w, f1b, f2w, f2b, f3w, f3b, f4w, f4b, f5w, f5b,
                  o_ref):
    m = parts_ref[0, :N_VARS] + parts_ref[1, :N_VARS]
    hv = jnp.maximum(
        hv0_ref[...]
        + jnp.dot(m, wmv_ref[...], preferred_element_type=jnp.float32), 0.0)
    x = jnp.mean(hv, axis=0, keepdims=True)
    x = _ln(x, ln1g[...], ln1b[...])
    x = jnp.maximum(jnp.dot(x, f1w[...], preferred_element_type=jnp.float32)
                    + f1b[...], 0.0)
    x = _ln(x, ln2g[...], ln2b[...])
    x = jnp.maximum(jnp.dot(x, f2w[...], preferred_element_type=jnp.float32)
                    + f2b[...], 0.0)
    x = jnp.dot(x, f3w[...], preferred_element_type=jnp.float32) + f3b[...]
    x = jnp.maximum(_ln(x, ln3g[...], ln3b[...]), 0.0)
    x = jnp.maximum(jnp.dot(x, f4w[...], preferred_element_type=jnp.float32)
                    + f4b[...], 0.0)
    o_ref[...] = jnp.dot(x, f5w[...], preferred_element_type=jnp.float32) \
        + f5b[...]


def _final(hv0, parts, wmv, head):
    args = [hv0, parts, wmv] + head
    return pl.pallas_call(
        _final_kernel,
        out_shape=jax.ShapeDtypeStruct((1, 1), jnp.float32),
    )(*args)


def _sc_pass(table, gidx3, sidx3, e3d, zeros_tbl):
    mesh = plsc.VectorSubcoreMesh(core_axis_name="c", subcore_axis_name="s")
    rpt = ROWS_PER_TILE

    @functools.partial(
        pl.kernel,
        out_type=jax.ShapeDtypeStruct((NC, N_SEG_PAD, H), jnp.float32),
        mesh=mesh,
        scratch_types=[
            pltpu.VMEM((ROWS_PER_W, EB), jnp.int32),
            pltpu.VMEM((ROWS_PER_W, EB), jnp.int32),
            pltpu.VMEM((EB, H), jnp.float32),
            pltpu.VMEM((EB, H), jnp.float32),
            pltpu.VMEM((EB, H), jnp.float32),
            pltpu.VMEM((EB, H), jnp.float32),
            pltpu.VMEM_SHARED((N_SEG_PAD, H), jnp.float32),
            pltpu.SemaphoreType.DMA,
            pltpu.SemaphoreType.DMA,
            pltpu.SemaphoreType.DMA,
            pltpu.SemaphoreType.DMA,
        ],
        compiler_params=pltpu.CompilerParams(use_tc_tiling_on_sc=False),
    )
    def k(table_hbm, gidx_hbm, sidx_hbm, e_hbm, zeros_hbm, out_hbm,
          gi_all, si_all, r0, r1, e0, e1, acc, sg0, sg1, se0, se1):
        cid = lax.axis_index("c")
        sid = lax.axis_index("s")
        wid = cid * NS + sid
        base = wid * ROWS_PER_W
        nblk = jnp.minimum(ROWS_PER_W, N_BLK - base)
        rbufs = (r0, r1)
        ebufs = (e0, e1)
        gsems = (sg0, sg1)
        esems = (se0, se1)

        pltpu.sync_copy(zeros_hbm.at[pl.ds(sid * rpt, rpt)],
                        acc.at[pl.ds(sid * rpt, rpt)])
        pltpu.sync_copy(gidx_hbm.at[wid], gi_all)
        pltpu.sync_copy(sidx_hbm.at[wid], si_all)
        plsc.subcore_barrier()

        def issue(j, b):
            pltpu.async_copy(e_hbm.at[base + j], ebufs[b], esems[b])
            pltpu.async_copy(table_hbm.at[gi_all.at[j]], rbufs[b], gsems[b])

        def crunch(j, b):
            rb, eb = rbufs[b], ebufs[b]
            pltpu.make_async_copy(e_hbm.at[base + j], eb, esems[b]).wait()
            pltpu.make_async_copy(
                table_hbm.at[gi_all.at[j]], rb, gsems[b]).wait()

            @plsc.parallel_loop(0, EB, unroll=4)
            def mul_row(q):
                for c in range(H // 16):
                    s = pl.ds(c * 16, 16)
                    rb[q, s] = rb[q, s] * eb[q, s]

            pltpu.sync_copy(rb, acc.at[si_all.at[j]], add=True)

        @pl.when(0 < nblk)
        def _():
            issue(0, 0)

        @pl.when(1 < nblk)
        def _():
            issue(1, 1)

        @pl.loop(0, ROWS_PER_W, step=2)
        def _(t):
            for b in range(2):
                j = t + b

                @pl.when(j < nblk)
                def _():
                    crunch(j, b)

                @pl.when(j + 2 < nblk)
                def _():
                    issue(j + 2, b)

        plsc.subcore_barrier()
        pltpu.sync_copy(acc.at[pl.ds(sid * rpt, rpt)],
                        out_hbm.at[cid, pl.ds(sid * rpt, rpt)])

    return k(table, gidx3, sidx3, e3d, zeros_tbl)


def _pad_rows(x, n_rows):
    return jnp.concatenate(
        [x, jnp.zeros((n_rows - x.shape[0],) + x.shape[1:], x.dtype)])


def kernel(constraint_features, edge_index, edge_attr, variable_features,
           Wc, bc, Wv, bv, We, be, Wmc, Wmv,
           ln1_g, ln1_b, ln2_g, ln2_b, ln3_g, ln3_b,
           fc1_w, fc1_b, fc2_w, fc2_b, fc3_w, fc3_b,
           fc4_w, fc4_b, fc5_w, fc5_b):
    src_p = _pad_rows(edge_index[0].astype(jnp.int32), E_PAD) \
        .reshape(NW, ROWS_PER_W, EB)
    dst_p = _pad_rows(edge_index[1].astype(jnp.int32), E_PAD) \
        .reshape(NW, ROWS_PER_W, EB)
    zeros_tbl = jnp.zeros((N_SEG_PAD, H), jnp.float32)

    h_v0 = _node_embed(variable_features, Wv, bv)
    e = _edge_embed(edge_attr, We, be).reshape(N_BLK, EB, H)

    msgc_parts = _sc_pass(h_v0, dst_p, src_p, e, zeros_tbl)
    h_c = _update(constraint_features, Wc, bc, msgc_parts, Wmc)

    msgv_parts = _sc_pass(h_c, src_p, dst_p, e, zeros_tbl)

    head = [ln1_g.reshape(1, H), ln1_b.reshape(1, H),
            ln2_g.reshape(1, 128), ln2_b.reshape(1, 128),
            ln3_g.reshape(1, 256), ln3_b.reshape(1, 256),
            fc1_w, fc1_b.reshape(1, 128), fc2_w, fc2_b.reshape(1, 128),
            fc3_w, fc3_b.reshape(1, 256), fc4_w, fc4_b.reshape(1, 128),
            fc5_w, fc5_b.reshape(1, 1)]
    out = _final(h_v0, msgv_parts, Wmv, head)
    return out.reshape(1)


# speedup vs baseline: 7.0080x; 1.1831x over previous
import functools

import jax
import jax.numpy as jnp
from jax import lax
from jax.experimental import pallas as pl
from jax.experimental.pallas import tpu as pltpu
from jax.experimental.pallas import tpu_sc as plsc

N_CONS = 10000
N_VARS = 10000
N_EDGES = 320000
D_FEAT = 128
D_EDGE = 16
H = 64

NC = 2
NS = 16
NW = NC * NS
EB = 128
N_BLK = N_EDGES // EB
N_ROWS = 2528
ROWS_PER_W = N_ROWS // NW
E_PAD = N_ROWS * EB
N_SEG_PAD = 10240
ROWS_PER_TILE = N_SEG_PAD // NS


def _node_embed_kernel(x_ref, w_ref, b_ref, o_ref):
    o_ref[...] = jnp.maximum(
        jnp.dot(x_ref[...], w_ref[...], preferred_element_type=jnp.float32)
        + b_ref[...], 0.0)


def _node_embed(x, w, b):
    n = x.shape[0]
    blk = n // 5
    return pl.pallas_call(
        _node_embed_kernel,
        grid=(5,),
        in_specs=[
            pl.BlockSpec((blk, D_FEAT), lambda i: (i, 0)),
            pl.BlockSpec((D_FEAT, H), lambda i: (0, 0)),
            pl.BlockSpec((1, H), lambda i: (0, 0)),
        ],
        out_specs=pl.BlockSpec((blk, H), lambda i: (i, 0)),
        out_shape=jax.ShapeDtypeStruct((N_SEG_PAD, H), jnp.float32),
    )(x, w, b.reshape(1, H))


_EPAIR = N_EDGES // 2
_EBLK = _EPAIR // 16


def _edge_embed_kernel(x_ref, w_ref, b_ref, o_ref):
    o_ref[...] = jnp.maximum(
        jnp.dot(x_ref[...], w_ref[...], preferred_element_type=jnp.float32)
        + b_ref[...], 0.0)


def _edge_embed(ea, we, be):
    ea2 = ea.reshape(_EPAIR, 2 * D_EDGE)
    w2 = jnp.zeros((2 * D_EDGE, 2 * H), jnp.float32)
    w2 = w2.at[:D_EDGE, :H].set(we).at[D_EDGE:, H:].set(we)
    b2 = jnp.concatenate([be, be]).reshape(1, 2 * H)
    return pl.pallas_call(
        _edge_embed_kernel,
        grid=(16,),
        in_specs=[
            pl.BlockSpec((_EBLK, 2 * D_EDGE), lambda i: (i, 0)),
            pl.BlockSpec((2 * D_EDGE, 2 * H), lambda i: (0, 0)),
            pl.BlockSpec((1, 2 * H), lambda i: (0, 0)),
        ],
        out_specs=pl.BlockSpec((_EBLK, 2 * H), lambda i: (i, 0)),
        out_shape=jax.ShapeDtypeStruct((_EPAIR, 2 * H), jnp.float32),
    )(ea2, w2, b2)


def _update_kernel(x_ref, w_ref, b_ref, parts_ref, wm_ref, o_ref):
    m = parts_ref[0] + parts_ref[1]
    h0 = jnp.dot(x_ref[...], w_ref[...], preferred_element_type=jnp.float32)
    h0 = jnp.maximum(h0 + b_ref[...], 0.0)
    upd = jnp.dot(m, wm_ref[...], preferred_element_type=jnp.float32)
    o_ref[...] = jnp.maximum(h0 + upd, 0.0)


def _update(x, w, b, parts, wm):
    n = x.shape[0]
    blk = n // 5
    return pl.pallas_call(
        _update_kernel,
        grid=(5,),
        in_specs=[
            pl.BlockSpec((blk, D_FEAT), lambda i: (i, 0)),
            pl.BlockSpec((D_FEAT, H), lambda i: (0, 0)),
            pl.BlockSpec((1, H), lambda i: (0, 0)),
            pl.BlockSpec((2, blk, H), lambda i: (0, i, 0)),
            pl.BlockSpec((H, H), lambda i: (0, 0)),
        ],
        out_specs=pl.BlockSpec((blk, H), lambda i: (i, 0)),
        out_shape=jax.ShapeDtypeStruct((N_SEG_PAD, H), jnp.float32),
    )(x, w, b.reshape(1, H), parts, wm)


def _ln(x, g, b, eps=1e-5):
    mu = jnp.mean(x, axis=-1, keepdims=True)
    d = x - mu
    var = jnp.mean(d * d, axis=-1, keepdims=True)
    return d * lax.rsqrt(var + eps) * g + b


def _final_kernel(hv0_ref, parts_ref, wmv_ref,
                  ln1g, ln1b, ln2g, ln2b, ln3g, ln3b,
                  f1w, f1b, f2w, f2b, f3w, f3b, f4w, f4b, f5w, f5b,
                  o_ref):
    m = parts_ref[0, :N_VARS] + parts_ref[1, :N_VARS]
    hv = jnp.maximum(
        hv0_ref[:N_VARS]
        + jnp.dot(m, wmv_ref[...], preferred_element_type=jnp.float32), 0.0)
    x = jnp.mean(hv, axis=0, keepdims=True)
    x = _ln(x, ln1g[...], ln1b[...])
    x = jnp.maximum(jnp.dot(x, f1w[...], preferred_element_type=jnp.float32)
                    + f1b[...], 0.0)
    x = _ln(x, ln2g[...], ln2b[...])
    x = jnp.maximum(jnp.dot(x, f2w[...], preferred_element_type=jnp.float32)
                    + f2b[...], 0.0)
    x = jnp.dot(x, f3w[...], preferred_element_type=jnp.float32) + f3b[...]
    x = jnp.maximum(_ln(x, ln3g[...], ln3b[...]), 0.0)
    x = jnp.maximum(jnp.dot(x, f4w[...], preferred_element_type=jnp.float32)
                    + f4b[...], 0.0)
    o_ref[...] = jnp.dot(x, f5w[...], preferred_element_type=jnp.float32) \
        + f5b[...]


def _final(hv0, parts, wmv, head):
    args = [hv0, parts, wmv] + head
    return pl.pallas_call(
        _final_kernel,
        out_shape=jax.ShapeDtypeStruct((1, 1), jnp.float32),
    )(*args)


def _sc_pass(table, gidx3, sidx3, e3d, zeros_tbl):
    mesh = plsc.VectorSubcoreMesh(core_axis_name="c", subcore_axis_name="s")
    rpt = ROWS_PER_TILE

    @functools.partial(
        pl.kernel,
        out_type=jax.ShapeDtypeStruct((NC, N_SEG_PAD, H), jnp.float32),
        mesh=mesh,
        scratch_types=[
            pltpu.VMEM((ROWS_PER_W, EB), jnp.int32),
            pltpu.VMEM((EB,), jnp.int32),
            pltpu.VMEM((EB,), jnp.int32),
            pltpu.VMEM((EB, H), jnp.float32),
            pltpu.VMEM((EB, H), jnp.float32),
            pltpu.VMEM((EB, H), jnp.float32),
            pltpu.VMEM((EB, H), jnp.float32),
            pltpu.VMEM_SHARED((N_SEG_PAD, H), jnp.float32),
            pltpu.VMEM_SHARED((N_SEG_PAD, H), jnp.float32),
            pltpu.SemaphoreType.DMA,
            pltpu.SemaphoreType.DMA,
            pltpu.SemaphoreType.DMA,
            pltpu.SemaphoreType.DMA,
            pltpu.SemaphoreType.DMA,
            pltpu.SemaphoreType.DMA,
        ],
        compiler_params=pltpu.CompilerParams(use_tc_tiling_on_sc=False),
    )
    def k(table_hbm, gidx_hbm, sidx_hbm, e_hbm, zeros_hbm, out_hbm,
          gi_all, si0, si1, r0, r1, e0, e1, acc, tbl,
          sg0, sg1, se0, se1, ss0, ss1):
        cid = lax.axis_index("c")
        sid = lax.axis_index("s")
        wid = cid * NS + sid
        base = wid * ROWS_PER_W
        nblk = jnp.minimum(ROWS_PER_W, N_BLK - base)
        rbufs = (r0, r1)
        ebufs = (e0, e1)
        sibufs = (si0, si1)
        gsems = (sg0, sg1)
        esems = (se0, se1)
        ssems = (ss0, ss1)

        pltpu.sync_copy(zeros_hbm.at[pl.ds(sid * rpt, rpt)],
                        acc.at[pl.ds(sid * rpt, rpt)])
        pltpu.sync_copy(table_hbm.at[pl.ds(sid * rpt, rpt)],
                        tbl.at[pl.ds(sid * rpt, rpt)])
        pltpu.sync_copy(gidx_hbm.at[wid], gi_all)
        plsc.subcore_barrier()

        def issue(j, b):
            pltpu.async_copy(e_hbm.at[base + j], ebufs[b], esems[b])
            pltpu.async_copy(sidx_hbm.at[wid, j], sibufs[b], ssems[b])
            pltpu.async_copy(tbl.at[gi_all.at[j]], rbufs[b], gsems[b])

        def crunch(j, b):
            rb, eb = rbufs[b], ebufs[b]
            pltpu.make_async_copy(e_hbm.at[base + j], eb, esems[b]).wait()
            pltpu.make_async_copy(
                tbl.at[gi_all.at[j]], rb, gsems[b]).wait()

            @plsc.parallel_loop(0, EB, unroll=4)
            def mul_row(q):
                for c in range(H // 16):
                    s = pl.ds(c * 16, 16)
                    rb[q, s] = rb[q, s] * eb[q, s]

            pltpu.make_async_copy(
                sidx_hbm.at[wid, j], sibufs[b], ssems[b]).wait()
            pltpu.sync_copy(rb, acc.at[sibufs[b]], add=True)

        @pl.when(0 < nblk)
        def _():
            issue(0, 0)

        @pl.when(1 < nblk)
        def _():
            issue(1, 1)

        @pl.loop(0, ROWS_PER_W, step=2)
        def _(t):
            for b in range(2):
                j = t + b

                @pl.when(j < nblk)
                def _():
                    crunch(j, b)

                @pl.when(j + 2 < nblk)
                def _():
                    issue(j + 2, b)

        plsc.subcore_barrier()
        pltpu.sync_copy(acc.at[pl.ds(sid * rpt, rpt)],
                        out_hbm.at[cid, pl.ds(sid * rpt, rpt)])

    return k(table, gidx3, sidx3, e3d, zeros_tbl)


def _pad_rows(x, n_rows):
    return jnp.concatenate(
        [x, jnp.zeros((n_rows - x.shape[0],) + x.shape[1:], x.dtype)])


def kernel(constraint_features, edge_index, edge_attr, variable_features,
           Wc, bc, Wv, bv, We, be, Wmc, Wmv,
           ln1_g, ln1_b, ln2_g, ln2_b, ln3_g, ln3_b,
           fc1_w, fc1_b, fc2_w, fc2_b, fc3_w, fc3_b,
           fc4_w, fc4_b, fc5_w, fc5_b):
    src_p = _pad_rows(edge_index[0].astype(jnp.int32), E_PAD) \
        .reshape(NW, ROWS_PER_W, EB)
    dst_p = _pad_rows(edge_index[1].astype(jnp.int32), E_PAD) \
        .reshape(NW, ROWS_PER_W, EB)
    zeros_tbl = jnp.zeros((N_SEG_PAD, H), jnp.float32)

    h_v0 = _node_embed(variable_features, Wv, bv)
    e = _edge_embed(edge_attr, We, be).reshape(N_BLK, EB, H)

    msgc_parts = _sc_pass(h_v0, dst_p, src_p, e, zeros_tbl)
    h_c = _update(constraint_features, Wc, bc, msgc_parts, Wmc)

    msgv_parts = _sc_pass(h_c, src_p, dst_p, e, zeros_tbl)

    head = [ln1_g.reshape(1, H), ln1_b.reshape(1, H),
            ln2_g.reshape(1, 128), ln2_b.reshape(1, 128),
            ln3_g.reshape(1, 256), ln3_b.reshape(1, 256),
            fc1_w, fc1_b.reshape(1, 128), fc2_w, fc2_b.reshape(1, 128),
            fc3_w, fc3_b.reshape(1, 256), fc4_w, fc4_b.reshape(1, 128),
            fc5_w, fc5_b.reshape(1, 1)]
    out = _final(h_v0, msgv_parts, Wmv, head)
    return out.reshape(1)


# speedup vs baseline: 7.1719x; 1.0234x over previous
import functools

import jax
import jax.numpy as jnp
from jax import lax
from jax.experimental import pallas as pl
from jax.experimental.pallas import tpu as pltpu
from jax.experimental.pallas import tpu_sc as plsc

N_CONS = 10000
N_VARS = 10000
N_EDGES = 320000
D_FEAT = 128
D_EDGE = 16
H = 64

NC = 2
NS = 16
NW = NC * NS
EB = 128
N_BLK = N_EDGES // EB
N_ROWS = 2528
ROWS_PER_W = N_ROWS // NW
E_PAD = N_ROWS * EB
N_SEG_PAD = 10240
ROWS_PER_TILE = N_SEG_PAD // NS


def _node_embed_kernel(x_ref, w_ref, b_ref, o_ref):
    o_ref[...] = jnp.maximum(
        jnp.dot(x_ref[...], w_ref[...], preferred_element_type=jnp.float32)
        + b_ref[...], 0.0)


def _node_embed(x, w, b):
    n = x.shape[0]
    blk = n // 5
    return pl.pallas_call(
        _node_embed_kernel,
        grid=(5,),
        in_specs=[
            pl.BlockSpec((blk, D_FEAT), lambda i: (i, 0)),
            pl.BlockSpec((D_FEAT, H), lambda i: (0, 0)),
            pl.BlockSpec((1, H), lambda i: (0, 0)),
        ],
        out_specs=pl.BlockSpec((blk, H), lambda i: (i, 0)),
        out_shape=jax.ShapeDtypeStruct((N_SEG_PAD, H), jnp.float32),
    )(x, w, b.reshape(1, H))


_EPAIR = N_EDGES // 2
_EBLK = _EPAIR // 16


def _edge_embed_kernel(x_ref, w_ref, b_ref, o_ref):
    o_ref[...] = jnp.maximum(
        jnp.dot(x_ref[...], w_ref[...], preferred_element_type=jnp.float32)
        + b_ref[...], 0.0)


def _edge_embed(ea, we, be):
    ea2 = ea.reshape(_EPAIR, 2 * D_EDGE)
    w2 = jnp.zeros((2 * D_EDGE, 2 * H), jnp.float32)
    w2 = w2.at[:D_EDGE, :H].set(we).at[D_EDGE:, H:].set(we)
    b2 = jnp.concatenate([be, be]).reshape(1, 2 * H)
    return pl.pallas_call(
        _edge_embed_kernel,
        grid=(16,),
        in_specs=[
            pl.BlockSpec((_EBLK, 2 * D_EDGE), lambda i: (i, 0)),
            pl.BlockSpec((2 * D_EDGE, 2 * H), lambda i: (0, 0)),
            pl.BlockSpec((1, 2 * H), lambda i: (0, 0)),
        ],
        out_specs=pl.BlockSpec((_EBLK, 2 * H), lambda i: (i, 0)),
        out_shape=jax.ShapeDtypeStruct((_EPAIR, 2 * H), jnp.float32),
    )(ea2, w2, b2)


def _update_kernel(x_ref, w_ref, b_ref, parts_ref, wm_ref, o_ref):
    m = parts_ref[0] + parts_ref[1]
    h0 = jnp.dot(x_ref[...], w_ref[...], preferred_element_type=jnp.float32)
    h0 = jnp.maximum(h0 + b_ref[...], 0.0)
    upd = jnp.dot(m, wm_ref[...], preferred_element_type=jnp.float32)
    o_ref[...] = jnp.maximum(h0 + upd, 0.0)


def _update(x, w, b, parts, wm):
    n = x.shape[0]
    blk = n // 5
    return pl.pallas_call(
        _update_kernel,
        grid=(5,),
        in_specs=[
            pl.BlockSpec((blk, D_FEAT), lambda i: (i, 0)),
            pl.BlockSpec((D_FEAT, H), lambda i: (0, 0)),
            pl.BlockSpec((1, H), lambda i: (0, 0)),
            pl.BlockSpec((2, blk, H), lambda i: (0, i, 0)),
            pl.BlockSpec((H, H), lambda i: (0, 0)),
        ],
        out_specs=pl.BlockSpec((blk, H), lambda i: (i, 0)),
        out_shape=jax.ShapeDtypeStruct((N_SEG_PAD, H), jnp.float32),
    )(x, w, b.reshape(1, H), parts, wm)


def _pad_idx_kernel(x_ref, o_ref):
    o_ref[:N_BLK, :] = x_ref[...]


def _pad_idx(idx):
    return pl.pallas_call(
        _pad_idx_kernel,
        out_shape=jax.ShapeDtypeStruct((N_ROWS, EB), jnp.int32),
    )(idx.reshape(N_BLK, EB))


def _ln(x, g, b, eps=1e-5):
    mu = jnp.mean(x, axis=-1, keepdims=True)
    d = x - mu
    var = jnp.mean(d * d, axis=-1, keepdims=True)
    return d * lax.rsqrt(var + eps) * g + b


def _final_kernel(hv0_ref, parts_ref, wmv_ref,
                  ln1g, ln1b, ln2g, ln2b, ln3g, ln3b,
                  f1w, f1b, f2w, f2b, f3w, f3b, f4w, f4b, f5w, f5b,
                  o_ref):
    m = parts_ref[0, :N_VARS] + parts_ref[1, :N_VARS]
    hv = jnp.maximum(
        hv0_ref[:N_VARS]
        + jnp.dot(m, wmv_ref[...], preferred_element_type=jnp.float32), 0.0)
    x = jnp.mean(hv, axis=0, keepdims=True)
    x = _ln(x, ln1g[...], ln1b[...])
    x = jnp.maximum(jnp.dot(x, f1w[...], preferred_element_type=jnp.float32)
                    + f1b[...], 0.0)
    x = _ln(x, ln2g[...], ln2b[...])
    x = jnp.maximum(jnp.dot(x, f2w[...], preferred_element_type=jnp.float32)
                    + f2b[...], 0.0)
    x = jnp.dot(x, f3w[...], preferred_element_type=jnp.float32) + f3b[...]
    x = jnp.maximum(_ln(x, ln3g[...], ln3b[...]), 0.0)
    x = jnp.maximum(jnp.dot(x, f4w[...], preferred_element_type=jnp.float32)
                    + f4b[...], 0.0)
    o_ref[...] = jnp.dot(x, f5w[...], preferred_element_type=jnp.float32) \
        + f5b[...]


def _final(hv0, parts, wmv, head):
    args = [hv0, parts, wmv] + head
    return pl.pallas_call(
        _final_kernel,
        out_shape=jax.ShapeDtypeStruct((1, 1), jnp.float32),
    )(*args)


def _sc_pass(table, gidx3, sidx2, e3d):
    mesh = plsc.VectorSubcoreMesh(core_axis_name="c", subcore_axis_name="s")
    rpt = ROWS_PER_TILE

    @functools.partial(
        pl.kernel,
        out_type=jax.ShapeDtypeStruct((NC, N_SEG_PAD, H), jnp.float32),
        mesh=mesh,
        scratch_types=[
            pltpu.VMEM((ROWS_PER_W, EB), jnp.int32),
            pltpu.VMEM((EB,), jnp.int32),
            pltpu.VMEM((EB,), jnp.int32),
            pltpu.VMEM((EB, H), jnp.float32),
            pltpu.VMEM((EB, H), jnp.float32),
            pltpu.VMEM((EB, H), jnp.float32),
            pltpu.VMEM((EB, H), jnp.float32),
            pltpu.VMEM_SHARED((N_SEG_PAD, H), jnp.float32),
            pltpu.VMEM_SHARED((N_SEG_PAD, H), jnp.float32),
            pltpu.SemaphoreType.DMA,
            pltpu.SemaphoreType.DMA,
            pltpu.SemaphoreType.DMA,
            pltpu.SemaphoreType.DMA,
            pltpu.SemaphoreType.DMA,
            pltpu.SemaphoreType.DMA,
        ],
        compiler_params=pltpu.CompilerParams(use_tc_tiling_on_sc=False),
    )
    def k(table_hbm, gidx_hbm, sidx_hbm, e_hbm, out_hbm,
          gi_all, si0, si1, r0, r1, e0, e1, acc, tbl,
          sg0, sg1, se0, se1, ss0, ss1):
        cid = lax.axis_index("c")
        sid = lax.axis_index("s")
        wid = cid * NS + sid
        base = wid * ROWS_PER_W
        nblk = jnp.minimum(ROWS_PER_W, N_BLK - base)
        rbufs = (r0, r1)
        ebufs = (e0, e1)
        sibufs = (si0, si1)
        gsems = (sg0, sg1)
        esems = (se0, se1)
        ssems = (ss0, ss1)
        gsrcs = (tbl, table_hbm)

        @plsc.parallel_loop(0, EB, unroll=4)
        def zero_row(q):
            for c in range(H // 16):
                r0[q, pl.ds(c * 16, 16)] = jnp.zeros((16,), jnp.float32)

        for z in range(rpt // EB):
            pltpu.sync_copy(r0, acc.at[pl.ds(sid * rpt + z * EB, EB)])
        pltpu.sync_copy(table_hbm.at[pl.ds(sid * rpt, rpt)],
                        tbl.at[pl.ds(sid * rpt, rpt)])
        pltpu.sync_copy(gidx_hbm.at[wid], gi_all)
        plsc.subcore_barrier()

        def issue(j, b):
            pltpu.async_copy(e_hbm.at[base + j], ebufs[b], esems[b])
            pltpu.async_copy(sidx_hbm.at[base + j], sibufs[b], ssems[b])
            pltpu.async_copy(gsrcs[b].at[gi_all.at[j]], rbufs[b], gsems[b])

        def crunch(j, b):
            rb, eb = rbufs[b], ebufs[b]
            pltpu.make_async_copy(e_hbm.at[base + j], eb, esems[b]).wait()
            pltpu.make_async_copy(
                gsrcs[b].at[gi_all.at[j]], rb, gsems[b]).wait()

            @plsc.parallel_loop(0, EB, unroll=4)
            def mul_row(q):
                for c in range(H // 16):
                    s = pl.ds(c * 16, 16)
                    rb[q, s] = rb[q, s] * eb[q, s]

            pltpu.make_async_copy(
                sidx_hbm.at[base + j], sibufs[b], ssems[b]).wait()
            pltpu.sync_copy(rb, acc.at[sibufs[b]], add=True)

        @pl.when(0 < nblk)
        def _():
            issue(0, 0)

        @pl.when(1 < nblk)
        def _():
            issue(1, 1)

        @pl.loop(0, ROWS_PER_W, step=2)
        def _(t):
            for b in range(2):
                j = t + b

                @pl.when(j < nblk)
                def _():
                    crunch(j, b)

                @pl.when(j + 2 < nblk)
                def _():
                    issue(j + 2, b)

        plsc.subcore_barrier()
        pltpu.sync_copy(acc.at[pl.ds(sid * rpt, rpt)],
                        out_hbm.at[cid, pl.ds(sid * rpt, rpt)])

    return k(table, gidx3, sidx2, e3d)


def kernel(constraint_features, edge_index, edge_attr, variable_features,
           Wc, bc, Wv, bv, We, be, Wmc, Wmv,
           ln1_g, ln1_b, ln2_g, ln2_b, ln3_g, ln3_b,
           fc1_w, fc1_b, fc2_w, fc2_b, fc3_w, fc3_b,
           fc4_w, fc4_b, fc5_w, fc5_b):
    src = edge_index[0].astype(jnp.int32)
    dst = edge_index[1].astype(jnp.int32)
    src_g = _pad_idx(src).reshape(NW, ROWS_PER_W, EB)
    dst_g = _pad_idx(dst).reshape(NW, ROWS_PER_W, EB)
    src_s = src.reshape(N_BLK, EB)
    dst_s = dst.reshape(N_BLK, EB)

    h_v0 = _node_embed(variable_features, Wv, bv)
    e = _edge_embed(edge_attr, We, be).reshape(N_BLK, EB, H)

    msgc_parts = _sc_pass(h_v0, dst_g, src_s, e)
    h_c = _update(constraint_features, Wc, bc, msgc_parts, Wmc)

    msgv_parts = _sc_pass(h_c, src_g, dst_s, e)

    head = [ln1_g.reshape(1, H), ln1_b.reshape(1, H),
            ln2_g.reshape(1, 128), ln2_b.reshape(1, 128),
            ln3_g.reshape(1, 256), ln3_b.reshape(1, 256),
            fc1_w, fc1_b.reshape(1, 128), fc2_w, fc2_b.reshape(1, 128),
            fc3_w, fc3_b.reshape(1, 256), fc4_w, fc4_b.reshape(1, 128),
            fc5_w, fc5_b.reshape(1, 1)]
    out = _final(h_v0, msgv_parts, Wmv, head)
    return out.reshape(1)


# speedup vs baseline: 7.5750x; 1.0562x over previous
import functools

import jax
import jax.numpy as jnp
from jax import lax
from jax.experimental import pallas as pl
from jax.experimental.pallas import tpu as pltpu
from jax.experimental.pallas import tpu_sc as plsc

N_CONS = 10000
N_VARS = 10000
N_EDGES = 320000
D_FEAT = 128
D_EDGE = 16
H = 64

NC = 2
NS = 16
NW = NC * NS
EB = 128
N_BLK = N_EDGES // EB
N_ROWS = 2528
ROWS_PER_W = N_ROWS // NW
E_PAD = N_ROWS * EB
N_SEG_PAD = 10240
ROWS_PER_TILE = N_SEG_PAD // NS


def _node_embed_kernel(x_ref, w_ref, b_ref, o_ref):
    o_ref[...] = jnp.maximum(
        jnp.dot(x_ref[...], w_ref[...], preferred_element_type=jnp.float32)
        + b_ref[...], 0.0)


def _node_embed(x, w, b):
    n = x.shape[0]
    blk = n // 5
    return pl.pallas_call(
        _node_embed_kernel,
        grid=(5,),
        in_specs=[
            pl.BlockSpec((blk, D_FEAT), lambda i: (i, 0)),
            pl.BlockSpec((D_FEAT, H), lambda i: (0, 0)),
            pl.BlockSpec((1, H), lambda i: (0, 0)),
        ],
        out_specs=pl.BlockSpec((blk, H), lambda i: (i, 0)),
        out_shape=jax.ShapeDtypeStruct((N_SEG_PAD, H), jnp.float32),
    )(x, w, b.reshape(1, H))


_EPAIR = N_EDGES // 2
_EBLK = _EPAIR // 16


def _edge_embed_kernel(x_ref, w_ref, b_ref, o_ref):
    y = jnp.maximum(
        jnp.dot(x_ref[...], w_ref[...], preferred_element_type=jnp.float32)
        + b_ref[...], 0.0).astype(jnp.bfloat16)
    o_ref[...] = pltpu.bitcast(y, jnp.int32)


def _edge_embed(ea, we, be):
    ea2 = ea.reshape(_EPAIR, 2 * D_EDGE)
    w2 = jnp.zeros((2 * D_EDGE, 2 * H), jnp.float32)
    w2 = w2.at[:D_EDGE, :H].set(we).at[D_EDGE:, H:].set(we)
    b2 = jnp.concatenate([be, be]).reshape(1, 2 * H)
    return pl.pallas_call(
        _edge_embed_kernel,
        grid=(16,),
        in_specs=[
            pl.BlockSpec((_EBLK, 2 * D_EDGE), lambda i: (i, 0)),
            pl.BlockSpec((2 * D_EDGE, 2 * H), lambda i: (0, 0)),
            pl.BlockSpec((1, 2 * H), lambda i: (0, 0)),
        ],
        out_specs=pl.BlockSpec((_EBLK // 2, 2 * H), lambda i: (i, 0)),
        out_shape=jax.ShapeDtypeStruct((_EPAIR // 2, 2 * H), jnp.int32),
    )(ea2, w2, b2)


def _update_kernel(x_ref, w_ref, b_ref, parts_ref, wm_ref, o_ref):
    m = parts_ref[0] + parts_ref[1]
    h0 = jnp.dot(x_ref[...], w_ref[...], preferred_element_type=jnp.float32)
    h0 = jnp.maximum(h0 + b_ref[...], 0.0)
    upd = jnp.dot(m, wm_ref[...], preferred_element_type=jnp.float32)
    o_ref[...] = jnp.maximum(h0 + upd, 0.0)


def _update(x, w, b, parts, wm):
    n = x.shape[0]
    blk = n // 5
    return pl.pallas_call(
        _update_kernel,
        grid=(5,),
        in_specs=[
            pl.BlockSpec((blk, D_FEAT), lambda i: (i, 0)),
            pl.BlockSpec((D_FEAT, H), lambda i: (0, 0)),
            pl.BlockSpec((1, H), lambda i: (0, 0)),
            pl.BlockSpec((2, blk, H), lambda i: (0, i, 0)),
            pl.BlockSpec((H, H), lambda i: (0, 0)),
        ],
        out_specs=pl.BlockSpec((blk, H), lambda i: (i, 0)),
        out_shape=jax.ShapeDtypeStruct((N_SEG_PAD, H), jnp.float32),
    )(x, w, b.reshape(1, H), parts, wm)


def _pad_idx_kernel(x_ref, o_ref):
    o_ref[:N_BLK, :] = x_ref[...]


def _pad_idx(idx):
    return pl.pallas_call(
        _pad_idx_kernel,
        out_shape=jax.ShapeDtypeStruct((N_ROWS, EB), jnp.int32),
    )(idx.reshape(N_BLK, EB))


def _ln(x, g, b, eps=1e-5):
    mu = jnp.mean(x, axis=-1, keepdims=True)
    d = x - mu
    var = jnp.mean(d * d, axis=-1, keepdims=True)
    return d * lax.rsqrt(var + eps) * g + b


def _final_kernel(hv0_ref, parts_ref, wmv_ref,
                  ln1g, ln1b, ln2g, ln2b, ln3g, ln3b,
                  f1w, f1b, f2w, f2b, f3w, f3b, f4w, f4b, f5w, f5b,
                  o_ref):
    m = parts_ref[0, :N_VARS] + parts_ref[1, :N_VARS]
    hv = jnp.maximum(
        hv0_ref[:N_VARS]
        + jnp.dot(m, wmv_ref[...], preferred_element_type=jnp.float32), 0.0)
    x = jnp.mean(hv, axis=0, keepdims=True)
    x = _ln(x, ln1g[...], ln1b[...])
    x = jnp.maximum(jnp.dot(x, f1w[...], preferred_element_type=jnp.float32)
                    + f1b[...], 0.0)
    x = _ln(x, ln2g[...], ln2b[...])
    x = jnp.maximum(jnp.dot(x, f2w[...], preferred_element_type=jnp.float32)
                    + f2b[...], 0.0)
    x = jnp.dot(x, f3w[...], preferred_element_type=jnp.float32) + f3b[...]
    x = jnp.maximum(_ln(x, ln3g[...], ln3b[...]), 0.0)
    x = jnp.maximum(jnp.dot(x, f4w[...], preferred_element_type=jnp.float32)
                    + f4b[...], 0.0)
    o_ref[...] = jnp.dot(x, f5w[...], preferred_element_type=jnp.float32) \
        + f5b[...]


def _final(hv0, parts, wmv, head):
    args = [hv0, parts, wmv] + head
    return pl.pallas_call(
        _final_kernel,
        out_shape=jax.ShapeDtypeStruct((1, 1), jnp.float32),
    )(*args)


def _sc_pass(table, gidx3, sidx2, e3d):
    mesh = plsc.VectorSubcoreMesh(core_axis_name="c", subcore_axis_name="s")
    rpt = ROWS_PER_TILE

    @functools.partial(
        pl.kernel,
        out_type=jax.ShapeDtypeStruct((NC, N_SEG_PAD, H), jnp.float32),
        mesh=mesh,
        scratch_types=[
            pltpu.VMEM((ROWS_PER_W, EB), jnp.int32),
            pltpu.VMEM((EB,), jnp.int32),
            pltpu.VMEM((EB,), jnp.int32),
            pltpu.VMEM((EB, H), jnp.float32),
            pltpu.VMEM((EB, H), jnp.float32),
            pltpu.VMEM((EB // 4, 2 * H), jnp.int32),
            pltpu.VMEM((EB // 4, 2 * H), jnp.int32),
            pltpu.VMEM_SHARED((N_SEG_PAD, H), jnp.float32),
            pltpu.VMEM_SHARED((N_SEG_PAD, H), jnp.float32),
            pltpu.SemaphoreType.DMA,
            pltpu.SemaphoreType.DMA,
            pltpu.SemaphoreType.DMA,
            pltpu.SemaphoreType.DMA,
            pltpu.SemaphoreType.DMA,
            pltpu.SemaphoreType.DMA,
        ],
        compiler_params=pltpu.CompilerParams(use_tc_tiling_on_sc=False),
    )
    def k(table_hbm, gidx_hbm, sidx_hbm, e_hbm, out_hbm,
          gi_all, si0, si1, r0, r1, e0, e1, acc, tbl,
          sg0, sg1, se0, se1, ss0, ss1):
        cid = lax.axis_index("c")
        sid = lax.axis_index("s")
        wid = cid * NS + sid
        base = wid * ROWS_PER_W
        nblk = jnp.minimum(ROWS_PER_W, N_BLK - base)
        rbufs = (r0, r1)
        ebufs = (e0, e1)
        sibufs = (si0, si1)
        gsems = (sg0, sg1)
        esems = (se0, se1)
        ssems = (ss0, ss1)
        gsrcs = (tbl, table_hbm)

        @plsc.parallel_loop(0, EB, unroll=4)
        def zero_row(q):
            for c in range(H // 16):
                r0[q, pl.ds(c * 16, 16)] = jnp.zeros((16,), jnp.float32)

        for z in range(rpt // EB):
            pltpu.sync_copy(r0, acc.at[pl.ds(sid * rpt + z * EB, EB)])
        pltpu.sync_copy(table_hbm.at[pl.ds(sid * rpt, rpt)],
                        tbl.at[pl.ds(sid * rpt, rpt)])
        pltpu.sync_copy(gidx_hbm.at[wid], gi_all)
        plsc.subcore_barrier()

        def issue(j, b):
            pltpu.async_copy(e_hbm.at[base + j], ebufs[b], esems[b])
            pltpu.async_copy(sidx_hbm.at[base + j], sibufs[b], ssems[b])
            pltpu.async_copy(gsrcs[b].at[gi_all.at[j]], rbufs[b], gsems[b])

        def crunch(j, b):
            rb, eb = rbufs[b], ebufs[b]
            pltpu.make_async_copy(e_hbm.at[base + j], eb, esems[b]).wait()
            pltpu.make_async_copy(
                gsrcs[b].at[gi_all.at[j]], rb, gsems[b]).wait()

            @plsc.parallel_loop(0, EB // 4, unroll=4)
            def mul_row(g):
                for h in range(2):
                    for c in range(H // 16):
                        w = eb[g, pl.ds(h * H + c * 16, 16)]
                        lo = lax.bitcast_convert_type(w << 16, jnp.float32)
                        hi = lax.bitcast_convert_type(
                            (w >> 16) << 16, jnp.float32)
                        s = pl.ds(c * 16, 16)
                        rb[4 * g + h, s] = rb[4 * g + h, s] * lo
                        rb[4 * g + h + 2, s] = rb[4 * g + h + 2, s] * hi

            pltpu.make_async_copy(
                sidx_hbm.at[base + j], sibufs[b], ssems[b]).wait()
            pltpu.sync_copy(rb, acc.at[sibufs[b]], add=True)

        @pl.when(0 < nblk)
        def _():
            issue(0, 0)

        @pl.when(1 < nblk)
        def _():
            issue(1, 1)

        @pl.loop(0, ROWS_PER_W, step=2)
        def _(t):
            for b in range(2):
                j = t + b

                @pl.when(j < nblk)
                def _():
                    crunch(j, b)

                @pl.when(j + 2 < nblk)
                def _():
                    issue(j + 2, b)

        plsc.subcore_barrier()
        pltpu.sync_copy(acc.at[pl.ds(sid * rpt, rpt)],
                        out_hbm.at[cid, pl.ds(sid * rpt, rpt)])

    return k(table, gidx3, sidx2, e3d)


def kernel(constraint_features, edge_index, edge_attr, variable_features,
           Wc, bc, Wv, bv, We, be, Wmc, Wmv,
           ln1_g, ln1_b, ln2_g, ln2_b, ln3_g, ln3_b,
           fc1_w, fc1_b, fc2_w, fc2_b, fc3_w, fc3_b,
           fc4_w, fc4_b, fc5_w, fc5_b):
    src = edge_index[0].astype(jnp.int32)
    dst = edge_index[1].astype(jnp.int32)
    src_g = _pad_idx(src).reshape(NW, ROWS_PER_W, EB)
    dst_g = _pad_idx(dst).reshape(NW, ROWS_PER_W, EB)
    src_s = src.reshape(N_BLK, EB)
    dst_s = dst.reshape(N_BLK, EB)

    h_v0 = _node_embed(variable_features, Wv, bv)
    e = _edge_embed(edge_attr, We, be).reshape(N_BLK, EB // 4, 2 * H)

    msgc_parts = _sc_pass(h_v0, dst_g, src_s, e)
    h_c = _update(constraint_features, Wc, bc, msgc_parts, Wmc)

    msgv_parts = _sc_pass(h_c, src_g, dst_s, e)

    head = [ln1_g.reshape(1, H), ln1_b.reshape(1, H),
            ln2_g.reshape(1, 128), ln2_b.reshape(1, 128),
            ln3_g.reshape(1, 256), ln3_b.reshape(1, 256),
            fc1_w, fc1_b.reshape(1, 128), fc2_w, fc2_b.reshape(1, 128),
            fc3_w, fc3_b.reshape(1, 256), fc4_w, fc4_b.reshape(1, 128),
            fc5_w, fc5_b.reshape(1, 1)]
    out = _final(h_v0, msgv_parts, Wmv, head)
    return out.reshape(1)
